# baseline (speedup 1.0000x reference)
import functools

import jax
import jax.numpy as jnp
from jax import lax
from jax.experimental import pallas as pl
from jax.experimental.pallas import tpu as pltpu
from jax.experimental.pallas import tpu_sc as plsc

N = 10000
D = 128
G = 64
E = 320000
EPS = 1e-5
ISQ = float(1.0 / (1.0 + EPS) ** 0.5)

NP = 10240
PADIDX = 10100
K = 128
NC, NS = 2, 16
NW = NC * NS
NCH = 80
EPT = NCH * K
EPAD = NW * EPT
IB = 4
RB = 2
NGRP = NCH // IB
RPT = NP // NS
BR = 1024
GRID = NP // BR

_mesh = plsc.VectorSubcoreMesh(core_axis_name="c", subcore_axis_name="s")


@functools.partial(
    pl.kernel,
    mesh=_mesh,
    out_type=jax.ShapeDtypeStruct((NC, NP, D), jnp.float32),
    scratch_types=[
        pltpu.VMEM((IB, K), jnp.int32),
        pltpu.VMEM((IB, K), jnp.int32),
        pltpu.VMEM((RB, K, D), jnp.float32),
        pltpu.VMEM_SHARED((NP, D), jnp.float32),
    ] + [pltpu.SemaphoreType.DMA] * (2 * IB + RB),
)
def _edge_sc(h_hbm, src_hbm, dst_hbm, out_hbm, sidx, didx, rows, acc, *sems):
    ssem = sems[:IB]
    dsem = sems[IB:2 * IB]
    rsem = sems[2 * IB:]
    c = lax.axis_index("c")
    s = lax.axis_index("s")
    wid = s * NC + c
    r0 = s * RPT

    pltpu.sync_copy(h_hbm.at[pl.ds(r0, RPT)], acc.at[pl.ds(r0, RPT)])
    plsc.subcore_barrier()

    def idx_start(j, bi):
        pltpu.async_copy(src_hbm.at[wid, j], sidx.at[bi], ssem[bi])
        pltpu.async_copy(dst_hbm.at[wid, j], didx.at[bi], dsem[bi])

    def gather_start(j, bi, br):
        pltpu.make_async_copy(src_hbm.at[wid, j], sidx.at[bi], ssem[bi]).wait()
        pltpu.async_copy(h_hbm.at[sidx.at[bi]], rows.at[br], rsem[br])

    def finish(j, bi, br):
        pltpu.make_async_copy(h_hbm.at[sidx.at[bi]], rows.at[br],
                              rsem[br]).wait()
        pltpu.make_async_copy(dst_hbm.at[wid, j], didx.at[bi], dsem[bi]).wait()
        pltpu.sync_copy(rows.at[br], acc.at[didx.at[bi]], add=True)

    for b in range(IB):
        idx_start(b, b)
    for b in range(RB):
        gather_start(b, b, b)

    def body(g, carry):
        for u in range(IB):
            j = g * IB + u
            finish(j, u, u % RB)
            idx_start(j + IB, u)
            gather_start(j + RB, (u + RB) % IB, u % RB)
        return carry
    lax.fori_loop(0, NGRP - 1, body, 0)

    base = (NGRP - 1) * IB
    for u in range(IB):
        j = base + u
        finish(j, u, u % RB)
        if u < IB - RB:
            gather_start(j + RB, (u + RB) % IB, u % RB)

    plsc.subcore_barrier()
    pltpu.sync_copy(acc.at[pl.ds(r0, RPT)], out_hbm.at[c, pl.ds(r0, RPT)])


def _dinv_body(c0_ref, c1_ref, dinv_ref):
    dinv_ref[...] = lax.rsqrt(c0_ref[...] + c1_ref[...] - 1.0)


def _dinv_tc(agg):
    return pl.pallas_call(
        _dinv_body,
        grid=(GRID,),
        in_specs=[
            pl.BlockSpec((BR, D), lambda i: (i, 0)),
            pl.BlockSpec((BR, D), lambda i: (i, 0)),
        ],
        out_specs=pl.BlockSpec((BR, D), lambda i: (i, 0)),
        out_shape=jax.ShapeDtypeStruct((NP, D), jnp.float32),
    )(agg[0], agg[1])


def _mm_body(x_ref, w_ref, dinv_ref, o_ref):
    o_ref[...] = dinv_ref[...] * jnp.dot(
        x_ref[...], w_ref[...], preferred_element_type=jnp.float32)


def _mm_tc(xp, w, dinv_col):
    return pl.pallas_call(
        _mm_body,
        grid=(GRID,),
        in_specs=[
            pl.BlockSpec((BR, D), lambda i: (i, 0)),
            pl.BlockSpec((D, D), lambda i: (0, 0)),
            pl.BlockSpec((BR, D), lambda i: (i, 0)),
        ],
        out_specs=pl.BlockSpec((BR, D), lambda i: (i, 0)),
        out_shape=jax.ShapeDtypeStruct((NP, D), jnp.float32),
    )(xp, w, dinv_col)


def _layer_body(a0_ref, a1_ref, hp_ref, dinv_ref, b_ref, g_ref, be_ref,
                w_ref, o_ref):
    t = dinv_ref[...] * (a0_ref[...] + a1_ref[...] - hp_ref[...]) + b_ref[...]
    t = t * (g_ref[...] * ISQ) + be_ref[...]
    t = jnp.maximum(t, 0.0)
    o_ref[...] = dinv_ref[...] * jnp.dot(
        t, w_ref[...], preferred_element_type=jnp.float32)


def _layer_tc(agg, hp, dinv_col, b, g, be, wn):
    return pl.pallas_call(
        _layer_body,
        grid=(GRID,),
        in_specs=[
            pl.BlockSpec((BR, D), lambda i: (i, 0)),
            pl.BlockSpec((BR, D), lambda i: (i, 0)),
            pl.BlockSpec((BR, D), lambda i: (i, 0)),
            pl.BlockSpec((BR, D), lambda i: (i, 0)),
            pl.BlockSpec((1, D), lambda i: (0, 0)),
            pl.BlockSpec((1, D), lambda i: (0, 0)),
            pl.BlockSpec((1, D), lambda i: (0, 0)),
            pl.BlockSpec((D, D), lambda i: (0, 0)),
        ],
        out_specs=pl.BlockSpec((BR, D), lambda i: (i, 0)),
        out_shape=jax.ShapeDtypeStruct((NP, D), jnp.float32),
    )(agg[0], agg[1], hp, dinv_col, b, g, be, wn)


def _final_body(a0_ref, a1_ref, hp_ref, dinv_ref, b_ref, bt_ref,
                h_ref, hg_ref, accs, cnts):
    i = pl.program_id(0)
    h3 = dinv_ref[...] * (a0_ref[...] + a1_ref[...] - hp_ref[...]) + b_ref[...]
    h_ref[...] = h3
    ids = lax.broadcasted_iota(jnp.int32, (BR, D), 1)
    oh = (bt_ref[...] == ids).astype(jnp.float32)
    part = lax.dot_general(oh, h3, (((0,), (0,)), ((), ())),
                           preferred_element_type=jnp.float32)
    cpart = lax.dot_general(oh, jnp.ones((BR, D), jnp.float32),
                            (((0,), (0,)), ((), ())),
                            preferred_element_type=jnp.float32)

    @pl.when(i == 0)
    def _():
        accs[...] = jnp.zeros_like(accs)
        cnts[...] = jnp.zeros_like(cnts)

    accs[...] += part
    cnts[...] += cpart

    @pl.when(i == GRID - 1)
    def _():
        hg_ref[...] = (accs[...] / jnp.maximum(cnts[...], 1.0))[:G, :]


def _final_tc(agg, hp, dinv_mat, b, batch_rep):
    return pl.pallas_call(
        _final_body,
        grid=(GRID,),
        in_specs=[
            pl.BlockSpec((BR, D), lambda i: (i, 0)),
            pl.BlockSpec((BR, D), lambda i: (i, 0)),
            pl.BlockSpec((BR, D), lambda i: (i, 0)),
            pl.BlockSpec((BR, D), lambda i: (i, 0)),
            pl.BlockSpec((1, D), lambda i: (0, 0)),
            pl.BlockSpec((BR, D), lambda i: (i, 0)),
        ],
        out_specs=[
            pl.BlockSpec((BR, D), lambda i: (i, 0)),
            pl.BlockSpec((G, D), lambda i: (0, 0)),
        ],
        out_shape=[
            jax.ShapeDtypeStruct((NP, D), jnp.float32),
            jax.ShapeDtypeStruct((G, D), jnp.float32),
        ],
        scratch_shapes=[
            pltpu.VMEM((D, D), jnp.float32),
            pltpu.VMEM((D, D), jnp.float32),
        ],
    )(agg[0], agg[1], hp, dinv_mat, b, batch_rep)


def kernel(x, edge_index, batch, W1, b1, g1, be1, W2, b2, g2, be2, W3, b3):
    src = edge_index[0]
    dst = edge_index[1]
    pad = jnp.full((EPAD - E,), PADIDX, jnp.int32)
    srcp = jnp.concatenate([src.astype(jnp.int32), pad]).reshape(NW, NCH, K)
    dstp = jnp.concatenate([dst.astype(jnp.int32), pad]).reshape(NW, NCH, K)
    xp = jnp.pad(x, ((0, NP - N), (0, 0)))
    batchp = jnp.concatenate(
        [batch.astype(jnp.int32), jnp.full((NP - N,), 127, jnp.int32)])
    batch_rep = jnp.broadcast_to(batchp[:, None], (NP, D))
    b1r, g1r, be1r = b1.reshape(1, D), g1.reshape(1, D), be1.reshape(1, D)
    b2r, g2r, be2r = b2.reshape(1, D), g2.reshape(1, D), be2.reshape(1, D)
    b3r = b3.reshape(1, D)

    ag0 = _edge_sc(jnp.ones((NP, D), jnp.float32), dstp, dstp)
    dinv_col = _dinv_tc(ag0)

    h1 = _mm_tc(xp, W1, dinv_col)
    a1 = _edge_sc(h1, srcp, dstp)
    h2 = _layer_tc(a1, h1, dinv_col, b1r, g1r, be1r, W2)
    a2 = _edge_sc(h2, srcp, dstp)
    h3 = _layer_tc(a2, h2, dinv_col, b2r, g2r, be2r, W3)
    a3 = _edge_sc(h3, srcp, dstp)
    h_full, hg = _final_tc(a3, h3, dinv_col, b3r, batch_rep)
    return h_full[:N], hg


# speedup vs baseline: 3.6381x; 3.6381x over previous
import functools

import jax
import jax.numpy as jnp
from jax import lax
from jax.experimental import pallas as pl
from jax.experimental.pallas import tpu as pltpu
from jax.experimental.pallas import tpu_sc as plsc

N = 10000
D = 128
G = 64
E = 320000
EPS = 1e-5
ISQ = float(1.0 / (1.0 + EPS) ** 0.5)

NP = 10240
K = 128
NC, NS = 2, 16
NW = NC * NS
NCH = 80
EPT = NCH * K
EPAD = NW * EPT
IB = 4
RB = 2
NGRP = NCH // IB
RPT = NP // NS
BR = 1024
GRID = NP // BR

_mesh = plsc.VectorSubcoreMesh(core_axis_name="c", subcore_axis_name="s")


@functools.partial(
    pl.kernel,
    mesh=_mesh,
    out_type=jax.ShapeDtypeStruct((NC, NP, D), jnp.float32),
    scratch_types=[
        pltpu.VMEM((IB, K), jnp.int32),
        pltpu.VMEM((IB, K), jnp.int32),
        pltpu.VMEM((RB, K, D), jnp.float32),
        pltpu.VMEM_SHARED((NP, D), jnp.float32),
    ] + [pltpu.SemaphoreType.DMA] * (2 * IB + RB),
)
def _edge_sc(h_hbm, src_hbm, dst_hbm, out_hbm, sidx, didx, rows, acc, *sems):
    ssem = sems[:IB]
    dsem = sems[IB:2 * IB]
    rsem = sems[2 * IB:]
    c = lax.axis_index("c")
    s = lax.axis_index("s")
    wid = s * NC + c
    r0 = s * RPT

    pltpu.sync_copy(h_hbm.at[pl.ds(r0, RPT)], acc.at[pl.ds(r0, RPT)])
    plsc.subcore_barrier()

    def idx_start(j, bi):
        pltpu.async_copy(src_hbm.at[wid, j], sidx.at[bi], ssem[bi])
        pltpu.async_copy(dst_hbm.at[wid, j], didx.at[bi], dsem[bi])

    def gather_start(j, bi, br):
        pltpu.make_async_copy(src_hbm.at[wid, j], sidx.at[bi], ssem[bi]).wait()
        pltpu.async_copy(h_hbm.at[sidx.at[bi]], rows.at[br], rsem[br])

    def finish(j, bi, br):
        pltpu.make_async_copy(h_hbm.at[sidx.at[bi]], rows.at[br],
                              rsem[br]).wait()
        pltpu.make_async_copy(dst_hbm.at[wid, j], didx.at[bi], dsem[bi]).wait()
        pltpu.sync_copy(rows.at[br], acc.at[didx.at[bi]], add=True)

    for b in range(IB):
        idx_start(b, b)
    for b in range(RB):
        gather_start(b, b, b)

    def body(g, carry):
        for u in range(IB):
            j = g * IB + u
            finish(j, u, u % RB)
            idx_start(j + IB, u)
            gather_start(j + RB, (u + RB) % IB, u % RB)
        return carry
    lax.fori_loop(0, NGRP - 1, body, 0)

    base = (NGRP - 1) * IB
    for u in range(IB):
        j = base + u
        finish(j, u, u % RB)
        if u < IB - RB:
            gather_start(j + RB, (u + RB) % IB, u % RB)

    plsc.subcore_barrier()
    pltpu.sync_copy(acc.at[pl.ds(r0, RPT)], out_hbm.at[c, pl.ds(r0, RPT)])


def _dinv_body(c0_ref, c1_ref, dinv_ref):
    dinv_ref[...] = lax.rsqrt(c0_ref[...] + c1_ref[...] - 1.0)


def _dinv_tc(agg):
    return pl.pallas_call(
        _dinv_body,
        grid=(GRID,),
        in_specs=[
            pl.BlockSpec((BR, D), lambda i: (i, 0)),
            pl.BlockSpec((BR, D), lambda i: (i, 0)),
        ],
        out_specs=pl.BlockSpec((BR, D), lambda i: (i, 0)),
        out_shape=jax.ShapeDtypeStruct((NP, D), jnp.float32),
    )(agg[0], agg[1])


def _mm_body(x_ref, w_ref, dinv_ref, o_ref):
    o_ref[...] = dinv_ref[...] * jnp.dot(
        x_ref[...], w_ref[...], preferred_element_type=jnp.float32)


def _mm_tc(xp, w, dinv_col):
    return pl.pallas_call(
        _mm_body,
        grid=(GRID,),
        in_specs=[
            pl.BlockSpec((BR, D), lambda i: (i, 0)),
            pl.BlockSpec((D, D), lambda i: (0, 0)),
            pl.BlockSpec((BR, D), lambda i: (i, 0)),
        ],
        out_specs=pl.BlockSpec((BR, D), lambda i: (i, 0)),
        out_shape=jax.ShapeDtypeStruct((NP, D), jnp.float32),
    )(xp, w, dinv_col)


def _layer_body(a0_ref, a1_ref, hp_ref, dinv_ref, b_ref, g_ref, be_ref,
                w_ref, o_ref):
    t = dinv_ref[...] * (a0_ref[...] + a1_ref[...] - hp_ref[...]) + b_ref[...]
    t = t * (g_ref[...] * ISQ) + be_ref[...]
    t = jnp.maximum(t, 0.0)
    o_ref[...] = dinv_ref[...] * jnp.dot(
        t, w_ref[...], preferred_element_type=jnp.float32)


def _layer_tc(agg, hp, dinv_col, b, g, be, wn):
    return pl.pallas_call(
        _layer_body,
        grid=(GRID,),
        in_specs=[
            pl.BlockSpec((BR, D), lambda i: (i, 0)),
            pl.BlockSpec((BR, D), lambda i: (i, 0)),
            pl.BlockSpec((BR, D), lambda i: (i, 0)),
            pl.BlockSpec((BR, D), lambda i: (i, 0)),
            pl.BlockSpec((1, D), lambda i: (0, 0)),
            pl.BlockSpec((1, D), lambda i: (0, 0)),
            pl.BlockSpec((1, D), lambda i: (0, 0)),
            pl.BlockSpec((D, D), lambda i: (0, 0)),
        ],
        out_specs=pl.BlockSpec((BR, D), lambda i: (i, 0)),
        out_shape=jax.ShapeDtypeStruct((NP, D), jnp.float32),
    )(agg[0], agg[1], hp, dinv_col, b, g, be, wn)


def _final_body(a0_ref, a1_ref, hp_ref, dinv_ref, b_ref, bt_ref,
                h_ref, hg_ref, accs, cnts):
    i = pl.program_id(0)
    h3 = dinv_ref[...] * (a0_ref[...] + a1_ref[...] - hp_ref[...]) + b_ref[...]
    h_ref[...] = h3
    ids = lax.broadcasted_iota(jnp.int32, (BR, D), 1)
    oh = (bt_ref[...] == ids).astype(jnp.float32)
    part = lax.dot_general(oh, h3, (((0,), (0,)), ((), ())),
                           preferred_element_type=jnp.float32)
    cpart = lax.dot_general(oh, jnp.ones((BR, D), jnp.float32),
                            (((0,), (0,)), ((), ())),
                            preferred_element_type=jnp.float32)

    @pl.when(i == 0)
    def _():
        accs[...] = jnp.zeros_like(accs)
        cnts[...] = jnp.zeros_like(cnts)

    accs[...] += part
    cnts[...] += cpart

    @pl.when(i == GRID - 1)
    def _():
        hg_ref[...] = (accs[...] / jnp.maximum(cnts[...], 1.0))[:G, :]


def _final_tc(agg, hp, dinv_mat, b, batch_rep):
    return pl.pallas_call(
        _final_body,
        grid=(GRID,),
        in_specs=[
            pl.BlockSpec((BR, D), lambda i: (i, 0)),
            pl.BlockSpec((BR, D), lambda i: (i, 0)),
            pl.BlockSpec((BR, D), lambda i: (i, 0)),
            pl.BlockSpec((BR, D), lambda i: (i, 0)),
            pl.BlockSpec((1, D), lambda i: (0, 0)),
            pl.BlockSpec((BR, D), lambda i: (i, 0)),
        ],
        out_specs=[
            pl.BlockSpec((BR, D), lambda i: (i, 0)),
            pl.BlockSpec((G, D), lambda i: (0, 0)),
        ],
        out_shape=[
            jax.ShapeDtypeStruct((NP, D), jnp.float32),
            jax.ShapeDtypeStruct((G, D), jnp.float32),
        ],
        scratch_shapes=[
            pltpu.VMEM((D, D), jnp.float32),
            pltpu.VMEM((D, D), jnp.float32),
        ],
    )(agg[0], agg[1], hp, dinv_mat, b, batch_rep)


def kernel(x, edge_index, batch, W1, b1, g1, be1, W2, b2, g2, be2, W3, b3):
    src = edge_index[0]
    dst = edge_index[1]
    pi = jnp.arange(EPAD - E, dtype=jnp.int32)
    pad_src = pi % NP
    pad_dst = N + pi % (NP - N)
    srcp = jnp.concatenate([src.astype(jnp.int32), pad_src]).reshape(NW, NCH, K)
    dstp = jnp.concatenate([dst.astype(jnp.int32), pad_dst]).reshape(NW, NCH, K)
    xp = jnp.pad(x, ((0, NP - N), (0, 0)))
    batchp = jnp.concatenate(
        [batch.astype(jnp.int32), jnp.full((NP - N,), 127, jnp.int32)])
    batch_rep = jnp.broadcast_to(batchp[:, None], (NP, D))
    b1r, g1r, be1r = b1.reshape(1, D), g1.reshape(1, D), be1.reshape(1, D)
    b2r, g2r, be2r = b2.reshape(1, D), g2.reshape(1, D), be2.reshape(1, D)
    b3r = b3.reshape(1, D)

    ag0 = _edge_sc(jnp.ones((NP, D), jnp.float32), dstp, dstp)
    dinv_col = _dinv_tc(ag0)

    h1 = _mm_tc(xp, W1, dinv_col)
    a1 = _edge_sc(h1, srcp, dstp)
    h2 = _layer_tc(a1, h1, dinv_col, b1r, g1r, be1r, W2)
    a2 = _edge_sc(h2, srcp, dstp)
    h3 = _layer_tc(a2, h2, dinv_col, b2r, g2r, be2r, W3)
    a3 = _edge_sc(h3, srcp, dstp)
    h_full, hg = _final_tc(a3, h3, dinv_col, b3r, batch_rep)
    return h_full[:N], hg
